# Initial kernel scaffold; baseline (speedup 1.0000x reference)
import jax
import jax.numpy as jnp
from jax.experimental import pallas as pl


def kernel(w1, b1, t2, w2, g2, be2, t3, w3, g3, be3, t4, w4, g4, be4, w_fc, b_fc, x_nchw):
    raise NotImplementedError("write your pallas kernel here")


# speedup vs baseline: 2.0317x; 2.0317x over previous
import jax
import jax.numpy as jnp
from jax.experimental import pallas as pl
from jax.experimental.pallas import tpu as pltpu


_EPS = 1e-5
_SLOPE = 0.2
_B = 8


def _lrelu(v):
    return jnp.maximum(v, _SLOPE * v)


_TAP = {0: (1, -1), 1: (0, 0), 2: (1, 0), 3: (0, 1)}


def _shift_rows(z, d):
    if d == -1:
        return jnp.concatenate([jnp.zeros_like(z[:, :1]), z[:, :-1]], axis=1)
    if d == 1:
        return jnp.concatenate([z[:, 1:], jnp.zeros_like(z[:, :1])], axis=1)
    return z


def _shift_cols(z, d):
    if d == -1:
        return jnp.concatenate([jnp.zeros_like(z[:, :, :1]), z[:, :, :-1]], axis=2)
    if d == 1:
        return jnp.concatenate([z[:, :, 1:], jnp.zeros_like(z[:, :, :1])], axis=2)
    return z


def _conv_in_lrelu(y, B, H, W, w, g, be):
    Cin = y.shape[-1]
    Cout = w.shape[-1]
    Ho, Wo = H // 2, W // 2
    yb = y.astype(jnp.bfloat16).reshape(B, Ho, 2, Wo, 2, Cin)
    planes = {(p, q): yb[:, :, p, :, q, :] for p in (0, 1) for q in (0, 1)}
    cols = []
    for kh in range(4):
        p, di = _TAP[kh]
        for kw in range(4):
            q, dj = _TAP[kw]
            cols.append(_shift_cols(_shift_rows(planes[(p, q)], di), dj))
    xim = jnp.concatenate(cols, axis=-1).reshape(B * Ho * Wo, 16 * Cin)
    acc = jnp.dot(xim, w, preferred_element_type=jnp.float32)
    a = acc.reshape(B, Ho * Wo, Cout)
    mean = jnp.mean(a, axis=1, keepdims=True)
    var = jnp.mean(a * a, axis=1, keepdims=True) - mean * mean
    scale = g.reshape(1, 1, Cout) * jax.lax.rsqrt(var + _EPS)
    shift = be.reshape(1, 1, Cout) - mean * scale
    return _lrelu(a * scale + shift).reshape(B * Ho * Wo, Cout)


def _make_disc_kernel(B, H1, W1):
    M1 = H1 * W1

    def _disc_kernel(x1_ref, w1_ref, b1_ref, w2_ref, g2_ref, be2_ref,
                     w3_ref, g3_ref, be3_ref, w4_ref, g4_ref, be4_ref,
                     wfc_ref, bfc_ref, o_ref):
        y = jnp.dot(x1_ref[...].reshape(B * M1, x1_ref.shape[-1]), w1_ref[...],
                    preferred_element_type=jnp.float32)
        y = _lrelu(y + b1_ref[...])

        y = _conv_in_lrelu(y, B, H1, W1, w2_ref[...], g2_ref[...], be2_ref[...])
        y = _conv_in_lrelu(y, B, H1 // 2, W1 // 2, w3_ref[...], g3_ref[...], be3_ref[...])
        y = _conv_in_lrelu(y, B, H1 // 4, W1 // 4, w4_ref[...], g4_ref[...], be4_ref[...])

        M4, C4 = wfc_ref.shape
        z = jnp.sum(y.reshape(B, M4, C4) * wfc_ref[...][None], axis=1)
        z = jnp.sum(z, axis=1, keepdims=True) + bfc_ref[...]
        o_ref[...] = 0.5 * (jnp.tanh(0.5 * z) + 1.0)

    return _disc_kernel


def kernel(w1, b1, t2, w2, g2, be2, t3, w3, g3, be3, t4, w4, g4, be4,
           w_fc, b_fc, x_nchw):
    del t2, t3, t4
    N, Cin, H, W = x_nchw.shape
    Ho, Wo = H // 2, W // 2
    M1 = Ho * Wo
    B = _B if N % _B == 0 else 1

    x = jnp.transpose(x_nchw, (0, 2, 3, 1))
    xp = jnp.pad(x, ((0, 0), (1, 1), (1, 1), (0, 0)))
    taps = [xp[:, kh:kh + 2 * Ho:2, kw:kw + 2 * Wo:2, :]
            for kh in range(4) for kw in range(4)]
    x1 = jnp.concatenate(taps, axis=-1).reshape(N, M1, 16 * Cin)
    x1 = x1.astype(jnp.bfloat16)

    w2f = w2.reshape(-1, w2.shape[-1])
    w3f = w3.reshape(-1, w3.shape[-1])
    w4f = w4.reshape(-1, w4.shape[-1])

    in_specs = [
        pl.BlockSpec((B, M1, 16 * Cin), lambda n: (n, 0, 0)),
        pl.BlockSpec(w1.shape, lambda n: (0, 0)),
        pl.BlockSpec(b1.shape, lambda n: (0, 0)),
        pl.BlockSpec(w2f.shape, lambda n: (0, 0)),
        pl.BlockSpec(g2.shape, lambda n: (0, 0)),
        pl.BlockSpec(be2.shape, lambda n: (0, 0)),
        pl.BlockSpec(w3f.shape, lambda n: (0, 0)),
        pl.BlockSpec(g3.shape, lambda n: (0, 0)),
        pl.BlockSpec(be3.shape, lambda n: (0, 0)),
        pl.BlockSpec(w4f.shape, lambda n: (0, 0)),
        pl.BlockSpec(g4.shape, lambda n: (0, 0)),
        pl.BlockSpec(be4.shape, lambda n: (0, 0)),
        pl.BlockSpec(w_fc.shape, lambda n: (0, 0)),
        pl.BlockSpec(b_fc.shape, lambda n: (0, 0)),
    ]
    out = pl.pallas_call(
        _make_disc_kernel(B, Ho, Wo),
        out_shape=jax.ShapeDtypeStruct((N, 1), jnp.float32),
        grid=(N // B,),
        in_specs=in_specs,
        out_specs=pl.BlockSpec((B, 1), lambda n: (n, 0)),
        compiler_params=pltpu.CompilerParams(
            dimension_semantics=("parallel",),
            vmem_limit_bytes=48 * 1024 * 1024),
    )(x1, w1, b1, w2f, g2, be2, w3f, g3, be3, w4f, g4, be4, w_fc, b_fc)
    return out
